# Initial kernel scaffold; baseline (speedup 1.0000x reference)
import jax
import jax.numpy as jnp
from jax.experimental import pallas as pl


def kernel(x, edge_index, edge_attr, W1, a_src1, a_dst1, We1, ae1, b1, W2, a_src2, a_dst2, We2, ae2, b2):
    raise NotImplementedError("write your pallas kernel here")


# speedup vs baseline: 10.5315x; 10.5315x over previous
import functools

import jax
import jax.numpy as jnp
from jax import lax
from jax.experimental import pallas as pl
from jax.experimental.pallas import tpu as pltpu
from jax.experimental.pallas import tpu_sc as plsc

N = 10000
NP = 10240
E = 320000
D = 128
NC = 2
NS = 16
NW = NC * NS
EPW = E // NW
CH = 80
SB = 400
NCB = EPW // SB
NCC = SB // CH
NDR = (N + CH - 1) // CH

BLK = 2048
EBLK = 32768

_f32 = jnp.float32
_i32 = jnp.int32


def _k1a_body(x_ref, w_ref, as_ref, ad_ref, h_ref, s_ref, t_ref):
    h = jnp.dot(x_ref[...], w_ref[...], preferred_element_type=_f32)
    h_ref[...] = h
    s_ref[...] = jnp.sum(h * as_ref[...][None, :], axis=1)
    t_ref[...] = jnp.sum(h * ad_ref[...][None, :], axis=1)


_k1a = pl.pallas_call(
    _k1a_body,
    grid=(NP // BLK,),
    in_specs=[
        pl.BlockSpec((BLK, D), lambda i: (i, 0)),
        pl.BlockSpec((D, D), lambda i: (0, 0)),
        pl.BlockSpec((D,), lambda i: (0,)),
        pl.BlockSpec((D,), lambda i: (0,)),
    ],
    out_specs=[
        pl.BlockSpec((BLK, D), lambda i: (i, 0)),
        pl.BlockSpec((BLK,), lambda i: (i,)),
        pl.BlockSpec((BLK,), lambda i: (i,)),
    ],
    out_shape=[
        jax.ShapeDtypeStruct((NP, D), _f32),
        jax.ShapeDtypeStruct((NP,), _f32),
        jax.ShapeDtypeStruct((NP,), _f32),
    ],
)


E8 = E // 8
EBLK8 = 5000


def _k1b_body(ea_ref, w1_ref, w2_ref, u1_ref, u2_ref):
    ea = ea_ref[...]
    u1_ref[...] = jnp.dot(ea, w1_ref[...], preferred_element_type=_f32)
    u2_ref[...] = jnp.dot(ea, w2_ref[...], preferred_element_type=_f32)


_k1b = pl.pallas_call(
    _k1b_body,
    grid=(E8 // EBLK8,),
    in_specs=[
        pl.BlockSpec((EBLK8, D), lambda i: (i, 0)),
        pl.BlockSpec((D, 8), lambda i: (0, 0)),
        pl.BlockSpec((D, 8), lambda i: (0, 0)),
    ],
    out_specs=[
        pl.BlockSpec((EBLK8, 8), lambda i: (i, 0)),
        pl.BlockSpec((EBLK8, 8), lambda i: (i, 0)),
    ],
    out_shape=[
        jax.ShapeDtypeStruct((E8, 8), _f32),
        jax.ShapeDtypeStruct((E8, 8), _f32),
    ],
)


_mesh = plsc.VectorSubcoreMesh(
    core_axis_name="c", subcore_axis_name="s", num_cores=NC, num_subcores=NS
)

_sc_params = pltpu.CompilerParams(needs_layout_passes=False)


@functools.partial(
    pl.kernel,
    out_type=[
        jax.ShapeDtypeStruct((NP, D), _f32),
        jax.ShapeDtypeStruct((NP, D), _f32),
        jax.ShapeDtypeStruct((NW, NP), _f32),
    ],
    mesh=_mesh,
    scratch_types=[
        pltpu.VMEM((SB,), _i32),
        pltpu.VMEM((SB,), _i32),
        pltpu.VMEM((SB,), _f32),
        pltpu.VMEM((NP,), _f32),
        pltpu.VMEM((NP,), _f32),
        pltpu.VMEM((CH,), _i32),
        pltpu.VMEM((CH,), _i32),
        pltpu.VMEM((CH, D), _f32),
        pltpu.VMEM((1, NP), _f32),
        pltpu.VMEM_SHARED((N, D), _f32),
        pltpu.SemaphoreType.DMA,
    ],
    compiler_params=_sc_params,
)
def _l1(src_hbm, dst_hbm, u_hbm, s_hbm, t_hbm, h_hbm,
        numa, numb, dend,
        src_v, dst_v, u_v, s_v, t_v, idx_s, idx_d, rows, den_v,
        accn, sem):
    cc = lax.axis_index("c")
    sid = lax.axis_index("s")
    wid = sid * NC + cc
    iota = lax.iota(_i32, 16)
    zeros16 = jnp.zeros((16,), _f32)

    def _zrows(k, _):
        e = k * 16 + iota
        plsc.store_scatter(rows, [e // D, e % D], zeros16)
        return _
    lax.fori_loop(0, CH * D // 16, _zrows, None)

    def _zden(k, _):
        den_v[0, pl.ds(k * 16, 16)] = zeros16
        return _
    lax.fori_loop(0, NP // 16, _zden, None)

    for j in range(pl.cdiv(NDR, NS)):
        grp = sid + NS * j

        def _zz():
            pltpu.sync_copy(rows, accn.at[pl.ds(grp * CH, CH)])

        if (j + 1) * NS <= NDR:
            _zz()
        else:
            pl.when(grp < NDR)(_zz)
    plsc.subcore_barrier()

    pltpu.sync_copy(s_hbm, s_v)
    pltpu.sync_copy(t_hbm, t_v)

    base = wid * EPW

    def _block(b, _):
        boff = base + b * SB
        pltpu.sync_copy(src_hbm.at[pl.ds(boff, SB)], src_v)
        pltpu.sync_copy(dst_hbm.at[pl.ds(boff, SB)], dst_v)
        pltpu.sync_copy(u_hbm.at[pl.ds(boff, SB)], u_v)

        def _chunk(c, _):
            off = c * CH
            pltpu.sync_copy(src_hbm.at[pl.ds(boff + off, CH)], idx_s)
            pltpu.sync_copy(dst_hbm.at[pl.ds(boff + off, CH)], idx_d)
            pltpu.async_copy(h_hbm.at[idx_s], rows, sem).wait()
            for k in range(CH // 16):
                rid = k * 16 + iota
                id16 = idx_d[pl.ds(k * 16, 16)]
                sv = plsc.load_gather(s_v, [idx_s[pl.ds(k * 16, 16)]])
                tv = plsc.load_gather(t_v, [id16])
                uv = u_v[pl.ds(off + k * 16, 16)]
                a = sv + tv + uv
                a = jnp.where(a > 0, a, 0.2 * a)
                ex = jnp.exp(a)
                plsc.addupdate_scatter(den_v, [jnp.zeros((16,), _i32), id16], ex)

                def _feat(f, _):
                    fv = jnp.full((16,), f, _i32)
                    col = plsc.load_gather(rows, [rid, fv])
                    plsc.store_scatter(rows, [rid, fv], col * ex)
                    return _
                lax.fori_loop(0, D, _feat, None, unroll=8)
            pltpu.sync_copy(rows, accn.at[idx_d], add=True)
            return _
        lax.fori_loop(0, NCC, _chunk, None)
        return _
    lax.fori_loop(0, NCB, _block, None)

    pltpu.sync_copy(den_v, dend.at[pl.ds(wid, 1)])

    plsc.subcore_barrier()

    for j in range(pl.cdiv(NDR, NS)):
        grp = sid + NS * j

        def _drain():
            @pl.when(cc == 0)
            def _():
                pltpu.sync_copy(accn.at[pl.ds(grp * CH, CH)],
                                numa.at[pl.ds(grp * CH, CH)])

            @pl.when(cc == 1)
            def _():
                pltpu.sync_copy(accn.at[pl.ds(grp * CH, CH)],
                                numb.at[pl.ds(grp * CH, CH)])

        if (j + 1) * NS <= NDR:
            _drain()
        else:
            pl.when(grp < NDR)(_drain)


def _k4_body(na_ref, nb_ref, dd_ref, b1_ref, w2_ref, g_ref):
    num = na_ref[...] + nb_ref[...]
    den = jnp.sum(dd_ref[...], axis=0)
    h2 = jnp.maximum(num / (den[:, None] + 1e-16) + b1_ref[...][None, :], 0.0)
    g_ref[...] = jnp.sum(h2 * w2_ref[...][None, :], axis=1)


_k4 = pl.pallas_call(
    _k4_body,
    grid=(NP // BLK,),
    in_specs=[
        pl.BlockSpec((BLK, D), lambda i: (i, 0)),
        pl.BlockSpec((BLK, D), lambda i: (i, 0)),
        pl.BlockSpec((NW, BLK), lambda i: (0, i)),
        pl.BlockSpec((D,), lambda i: (0,)),
        pl.BlockSpec((D,), lambda i: (0,)),
    ],
    out_specs=[pl.BlockSpec((BLK,), lambda i: (i,))],
    out_shape=[jax.ShapeDtypeStruct((NP,), _f32)],
)


@functools.partial(
    pl.kernel,
    out_type=[
        jax.ShapeDtypeStruct((NW, NP), _f32),
        jax.ShapeDtypeStruct((NW, NP), _f32),
    ],
    mesh=_mesh,
    scratch_types=[
        pltpu.VMEM((EPW,), _i32),
        pltpu.VMEM((EPW,), _i32),
        pltpu.VMEM((EPW,), _f32),
        pltpu.VMEM((NP,), _f32),
        pltpu.VMEM((16,), _f32),
        pltpu.VMEM((1, NP), _f32),
        pltpu.VMEM((1, NP), _f32),
    ],
    compiler_params=_sc_params,
)
def _l2(src_hbm, dst_hbm, u_hbm, g_hbm, prm_hbm,
        num2, den2,
        src_v, dst_v, u_v, g_v, prm_v, num_v, den_v):
    cc = lax.axis_index("c")
    sid = lax.axis_index("s")
    wid = sid * NC + cc
    iota = lax.iota(_i32, 16)
    zeros16 = jnp.zeros((16,), _f32)

    def _zacc(k, _):
        num_v[0, pl.ds(k * 16, 16)] = zeros16
        den_v[0, pl.ds(k * 16, 16)] = zeros16
        return _
    lax.fori_loop(0, NP // 16, _zacc, None)

    base = wid * EPW
    pltpu.sync_copy(src_hbm.at[pl.ds(base, EPW)], src_v)
    pltpu.sync_copy(dst_hbm.at[pl.ds(base, EPW)], dst_v)
    pltpu.sync_copy(u_hbm.at[pl.ds(base, EPW)], u_v)
    pltpu.sync_copy(g_hbm, g_v)
    pltpu.sync_copy(prm_hbm, prm_v)
    a2s = plsc.load_gather(prm_v, [jnp.zeros((16,), _i32)])
    a2d = plsc.load_gather(prm_v, [jnp.ones((16,), _i32)])

    def _chunk(c, _):
        off = c * 16
        is16 = src_v[pl.ds(off, 16)]
        id16 = dst_v[pl.ds(off, 16)]
        gs = plsc.load_gather(g_v, [is16])
        gd = plsc.load_gather(g_v, [id16])
        a = a2s * gs + a2d * gd + u_v[pl.ds(off, 16)]
        a = jnp.where(a > 0, a, 0.2 * a)
        ex = jnp.exp(a)
        plsc.addupdate_scatter(num_v, [jnp.zeros((16,), _i32), id16], ex * gs)
        plsc.addupdate_scatter(den_v, [jnp.zeros((16,), _i32), id16], ex)
        return _
    lax.fori_loop(0, EPW // 16, _chunk, None, unroll=4)

    pltpu.sync_copy(num_v, num2.at[pl.ds(wid, 1)])
    pltpu.sync_copy(den_v, den2.at[pl.ds(wid, 1)])


def _k6_body(n2_ref, d2_ref, b2_ref, o_ref):
    num = jnp.sum(n2_ref[...], axis=0)
    den = jnp.sum(d2_ref[...], axis=0)
    o_ref[...] = jax.nn.sigmoid(num / (den + 1e-16) + b2_ref[0, 0])


_k6 = pl.pallas_call(
    _k6_body,
    grid=(NP // BLK,),
    in_specs=[
        pl.BlockSpec((NW, BLK), lambda i: (0, i)),
        pl.BlockSpec((NW, BLK), lambda i: (0, i)),
        pl.BlockSpec((1, 1), lambda i: (0, 0)),
    ],
    out_specs=[pl.BlockSpec((BLK,), lambda i: (i,))],
    out_shape=[jax.ShapeDtypeStruct((NP,), _f32)],
)


def kernel(x, edge_index, edge_attr,
           W1, a_src1, a_dst1, We1, ae1, b1,
           W2, a_src2, a_dst2, We2, ae2, b2):
    src = edge_index[0].astype(_i32)
    dst = edge_index[1].astype(_i32)
    xp = jnp.pad(x, ((0, NP - N), (0, 0)))
    we1 = (We1 @ ae1).astype(_f32)
    we2 = (We2[:, 0] * ae2[0]).astype(_f32)
    w81 = jnp.kron(jnp.eye(8, dtype=_f32), we1[:, None])
    w82 = jnp.kron(jnp.eye(8, dtype=_f32), we2[:, None])
    ea8 = edge_attr.reshape(E8, D)
    w2v = W2[:, 0].astype(_f32)
    prm2 = jnp.concatenate([a_src2, a_dst2, jnp.zeros((14,), _f32)])

    h, s, t = _k1a(xp, W1, a_src1, a_dst1)
    u18, u28 = _k1b(ea8, w81, w82)
    u1 = u18.reshape(E)
    u2 = u28.reshape(E)
    numa, numb, dend = _l1(src, dst, u1, s, t, h)
    g, = _k4(numa, numb, dend, b1, w2v)
    n2, d2 = _l2(src, dst, u2, g, prm2)
    out, = _k6(n2, d2, b2.reshape(1, 1))
    return out[:N]


# speedup vs baseline: 10.7303x; 1.0189x over previous
import functools

import jax
import jax.numpy as jnp
from jax import lax
from jax.experimental import pallas as pl
from jax.experimental.pallas import tpu as pltpu
from jax.experimental.pallas import tpu_sc as plsc

N = 10000
NP = 10240
E = 320000
D = 128
NC = 2
NS = 16
NW = NC * NS
EPW = E // NW
CH = 80
SB = 400
NCB = EPW // SB
NCC = SB // CH
NDR = (N + CH - 1) // CH

BLK = 2048
EBLK = 32768

_f32 = jnp.float32
_i32 = jnp.int32


def _k1a_body(x_ref, w_ref, as_ref, ad_ref, h_ref, s_ref, t_ref):
    h = jnp.dot(x_ref[...], w_ref[...], preferred_element_type=_f32)
    h_ref[...] = h
    s_ref[...] = jnp.sum(h * as_ref[...][None, :], axis=1)
    t_ref[...] = jnp.sum(h * ad_ref[...][None, :], axis=1)


_k1a = pl.pallas_call(
    _k1a_body,
    grid=(NP // BLK,),
    in_specs=[
        pl.BlockSpec((BLK, D), lambda i: (i, 0)),
        pl.BlockSpec((D, D), lambda i: (0, 0)),
        pl.BlockSpec((D,), lambda i: (0,)),
        pl.BlockSpec((D,), lambda i: (0,)),
    ],
    out_specs=[
        pl.BlockSpec((BLK, D), lambda i: (i, 0)),
        pl.BlockSpec((BLK,), lambda i: (i,)),
        pl.BlockSpec((BLK,), lambda i: (i,)),
    ],
    out_shape=[
        jax.ShapeDtypeStruct((NP, D), _f32),
        jax.ShapeDtypeStruct((NP,), _f32),
        jax.ShapeDtypeStruct((NP,), _f32),
    ],
)


E8 = E // 8
EBLK8 = 5000


def _k1b_body(ea_ref, w1_ref, w2_ref, u1_ref, u2_ref):
    ea = ea_ref[...]
    u1_ref[...] = jnp.dot(ea, w1_ref[...], preferred_element_type=_f32)
    u2_ref[...] = jnp.dot(ea, w2_ref[...], preferred_element_type=_f32)


_k1b = pl.pallas_call(
    _k1b_body,
    grid=(E8 // EBLK8,),
    in_specs=[
        pl.BlockSpec((EBLK8, D), lambda i: (i, 0)),
        pl.BlockSpec((D, 8), lambda i: (0, 0)),
        pl.BlockSpec((D, 8), lambda i: (0, 0)),
    ],
    out_specs=[
        pl.BlockSpec((EBLK8, 8), lambda i: (i, 0)),
        pl.BlockSpec((EBLK8, 8), lambda i: (i, 0)),
    ],
    out_shape=[
        jax.ShapeDtypeStruct((E8, 8), _f32),
        jax.ShapeDtypeStruct((E8, 8), _f32),
    ],
)


_mesh = plsc.VectorSubcoreMesh(
    core_axis_name="c", subcore_axis_name="s", num_cores=NC, num_subcores=NS
)

_sc_params = pltpu.CompilerParams(needs_layout_passes=False)


@functools.partial(
    pl.kernel,
    out_type=[
        jax.ShapeDtypeStruct((NP, D), _f32),
        jax.ShapeDtypeStruct((NP, D), _f32),
        jax.ShapeDtypeStruct((NW, NP), _f32),
    ],
    mesh=_mesh,
    scratch_types=[
        pltpu.VMEM((SB,), _f32),
        pltpu.VMEM((NP,), _f32),
        pltpu.VMEM((NP,), _f32),
        pltpu.VMEM((CH,), _i32),
        pltpu.VMEM((CH,), _i32),
        pltpu.VMEM((CH,), _f32),
        pltpu.VMEM((CH, D), _f32),
        pltpu.VMEM((1, NP), _f32),
        pltpu.VMEM_SHARED((N, D), _f32),
        pltpu.SemaphoreType.DMA,
    ],
    compiler_params=_sc_params,
)
def _l1(src_hbm, dst_hbm, u_hbm, s_hbm, t_hbm, h_hbm,
        numa, numb, dend,
        u_v, s_v, t_v, idx_s, idx_d, exq, rows, den_v,
        accn, sem):
    cc = lax.axis_index("c")
    sid = lax.axis_index("s")
    wid = sid * NC + cc
    iota = lax.iota(_i32, 16)
    zeros16 = jnp.zeros((16,), _f32)

    def _zrows(k, _):
        e = k * 16 + iota
        plsc.store_scatter(rows, [e // D, e % D], zeros16)
        return _
    lax.fori_loop(0, CH * D // 16, _zrows, None)

    def _zden(k, _):
        den_v[0, pl.ds(k * 16, 16)] = zeros16
        return _
    lax.fori_loop(0, NP // 16, _zden, None)

    for j in range(pl.cdiv(NDR, NS)):
        grp = sid + NS * j

        def _zz():
            pltpu.sync_copy(rows, accn.at[pl.ds(grp * CH, CH)])

        if (j + 1) * NS <= NDR:
            _zz()
        else:
            pl.when(grp < NDR)(_zz)
    plsc.subcore_barrier()

    pltpu.sync_copy(s_hbm, s_v)
    pltpu.sync_copy(t_hbm, t_v)

    base = wid * EPW

    def _block(b, _):
        boff = base + b * SB
        pltpu.sync_copy(u_hbm.at[pl.ds(boff, SB)], u_v)

        def _chunk(c, _):
            off = c * CH
            pltpu.sync_copy(src_hbm.at[pl.ds(boff + off, CH)], idx_s)
            pltpu.sync_copy(dst_hbm.at[pl.ds(boff + off, CH)], idx_d)
            cp = pltpu.async_copy(h_hbm.at[idx_s], rows, sem)
            for k in range(CH // 16):
                id16 = idx_d[pl.ds(k * 16, 16)]
                sv = plsc.load_gather(s_v, [idx_s[pl.ds(k * 16, 16)]])
                tv = plsc.load_gather(t_v, [id16])
                uv = u_v[pl.ds(off + k * 16, 16)]
                a = sv + tv + uv
                a = jnp.where(a > 0, a, 0.2 * a)
                ex = jnp.exp(a)
                exq[pl.ds(k * 16, 16)] = ex
                plsc.addupdate_scatter(den_v, [jnp.zeros((16,), _i32), id16], ex)
            cp.wait()
            for k in range(CH // 16):
                rid = k * 16 + iota
                ex = exq[pl.ds(k * 16, 16)]

                def _feat(f, _):
                    fv = jnp.full((16,), f, _i32)
                    col = plsc.load_gather(rows, [rid, fv])
                    plsc.store_scatter(rows, [rid, fv], col * ex)
                    return _
                lax.fori_loop(0, D, _feat, None, unroll=8)
            pltpu.sync_copy(rows, accn.at[idx_d], add=True)
            return _
        lax.fori_loop(0, NCC, _chunk, None)
        return _
    lax.fori_loop(0, NCB, _block, None)

    pltpu.sync_copy(den_v, dend.at[pl.ds(wid, 1)])

    plsc.subcore_barrier()

    for j in range(pl.cdiv(NDR, NS)):
        grp = sid + NS * j

        def _drain():
            @pl.when(cc == 0)
            def _():
                pltpu.sync_copy(accn.at[pl.ds(grp * CH, CH)],
                                numa.at[pl.ds(grp * CH, CH)])

            @pl.when(cc == 1)
            def _():
                pltpu.sync_copy(accn.at[pl.ds(grp * CH, CH)],
                                numb.at[pl.ds(grp * CH, CH)])

        if (j + 1) * NS <= NDR:
            _drain()
        else:
            pl.when(grp < NDR)(_drain)


def _k4_body(na_ref, nb_ref, dd_ref, b1_ref, w2_ref, g_ref):
    num = na_ref[...] + nb_ref[...]
    den = jnp.sum(dd_ref[...], axis=0)
    h2 = jnp.maximum(num / (den[:, None] + 1e-16) + b1_ref[...][None, :], 0.0)
    g_ref[...] = jnp.sum(h2 * w2_ref[...][None, :], axis=1)


_k4 = pl.pallas_call(
    _k4_body,
    grid=(NP // BLK,),
    in_specs=[
        pl.BlockSpec((BLK, D), lambda i: (i, 0)),
        pl.BlockSpec((BLK, D), lambda i: (i, 0)),
        pl.BlockSpec((NW, BLK), lambda i: (0, i)),
        pl.BlockSpec((D,), lambda i: (0,)),
        pl.BlockSpec((D,), lambda i: (0,)),
    ],
    out_specs=[pl.BlockSpec((BLK,), lambda i: (i,))],
    out_shape=[jax.ShapeDtypeStruct((NP,), _f32)],
)


@functools.partial(
    pl.kernel,
    out_type=[
        jax.ShapeDtypeStruct((NW, NP), _f32),
        jax.ShapeDtypeStruct((NW, NP), _f32),
    ],
    mesh=_mesh,
    scratch_types=[
        pltpu.VMEM((EPW,), _i32),
        pltpu.VMEM((EPW,), _i32),
        pltpu.VMEM((EPW,), _f32),
        pltpu.VMEM((NP,), _f32),
        pltpu.VMEM((16,), _f32),
        pltpu.VMEM((1, NP), _f32),
        pltpu.VMEM((1, NP), _f32),
    ],
    compiler_params=_sc_params,
)
def _l2(src_hbm, dst_hbm, u_hbm, g_hbm, prm_hbm,
        num2, den2,
        src_v, dst_v, u_v, g_v, prm_v, num_v, den_v):
    cc = lax.axis_index("c")
    sid = lax.axis_index("s")
    wid = sid * NC + cc
    iota = lax.iota(_i32, 16)
    zeros16 = jnp.zeros((16,), _f32)

    def _zacc(k, _):
        num_v[0, pl.ds(k * 16, 16)] = zeros16
        den_v[0, pl.ds(k * 16, 16)] = zeros16
        return _
    lax.fori_loop(0, NP // 16, _zacc, None)

    base = wid * EPW
    pltpu.sync_copy(src_hbm.at[pl.ds(base, EPW)], src_v)
    pltpu.sync_copy(dst_hbm.at[pl.ds(base, EPW)], dst_v)
    pltpu.sync_copy(u_hbm.at[pl.ds(base, EPW)], u_v)
    pltpu.sync_copy(g_hbm, g_v)
    pltpu.sync_copy(prm_hbm, prm_v)
    a2s = plsc.load_gather(prm_v, [jnp.zeros((16,), _i32)])
    a2d = plsc.load_gather(prm_v, [jnp.ones((16,), _i32)])

    def _chunk(c, _):
        off = c * 16
        is16 = src_v[pl.ds(off, 16)]
        id16 = dst_v[pl.ds(off, 16)]
        gs = plsc.load_gather(g_v, [is16])
        gd = plsc.load_gather(g_v, [id16])
        a = a2s * gs + a2d * gd + u_v[pl.ds(off, 16)]
        a = jnp.where(a > 0, a, 0.2 * a)
        ex = jnp.exp(a)
        plsc.addupdate_scatter(num_v, [jnp.zeros((16,), _i32), id16], ex * gs)
        plsc.addupdate_scatter(den_v, [jnp.zeros((16,), _i32), id16], ex)
        return _
    lax.fori_loop(0, EPW // 16, _chunk, None, unroll=4)

    pltpu.sync_copy(num_v, num2.at[pl.ds(wid, 1)])
    pltpu.sync_copy(den_v, den2.at[pl.ds(wid, 1)])


def _k6_body(n2_ref, d2_ref, b2_ref, o_ref):
    num = jnp.sum(n2_ref[...], axis=0)
    den = jnp.sum(d2_ref[...], axis=0)
    o_ref[...] = jax.nn.sigmoid(num / (den + 1e-16) + b2_ref[0, 0])


_k6 = pl.pallas_call(
    _k6_body,
    grid=(NP // BLK,),
    in_specs=[
        pl.BlockSpec((NW, BLK), lambda i: (0, i)),
        pl.BlockSpec((NW, BLK), lambda i: (0, i)),
        pl.BlockSpec((1, 1), lambda i: (0, 0)),
    ],
    out_specs=[pl.BlockSpec((BLK,), lambda i: (i,))],
    out_shape=[jax.ShapeDtypeStruct((NP,), _f32)],
)


def kernel(x, edge_index, edge_attr,
           W1, a_src1, a_dst1, We1, ae1, b1,
           W2, a_src2, a_dst2, We2, ae2, b2):
    src = edge_index[0].astype(_i32)
    dst = edge_index[1].astype(_i32)
    xp = jnp.pad(x, ((0, NP - N), (0, 0)))
    we1 = (We1 @ ae1).astype(_f32)
    we2 = (We2[:, 0] * ae2[0]).astype(_f32)
    w81 = jnp.kron(jnp.eye(8, dtype=_f32), we1[:, None])
    w82 = jnp.kron(jnp.eye(8, dtype=_f32), we2[:, None])
    ea8 = edge_attr.reshape(E8, D)
    w2v = W2[:, 0].astype(_f32)
    prm2 = jnp.concatenate([a_src2, a_dst2, jnp.zeros((14,), _f32)])

    h, s, t = _k1a(xp, W1, a_src1, a_dst1)
    u18, u28 = _k1b(ea8, w81, w82)
    u1 = u18.reshape(E)
    u2 = u28.reshape(E)
    numa, numb, dend = _l1(src, dst, u1, s, t, h)
    g, = _k4(numa, numb, dend, b1, w2v)
    n2, d2 = _l2(src, dst, u2, g, prm2)
    out, = _k6(n2, d2, b2.reshape(1, 1))
    return out[:N]


# speedup vs baseline: 11.1495x; 1.0391x over previous
import functools

import jax
import jax.numpy as jnp
from jax import lax
from jax.experimental import pallas as pl
from jax.experimental.pallas import tpu as pltpu
from jax.experimental.pallas import tpu_sc as plsc

N = 10000
NP = 10240
E = 320000
D = 128
NC = 2
NS = 16
NW = NC * NS
EPW = E // NW
CH = 80
SB = 400
NCB = EPW // SB
NCC = SB // CH
NDR = (N + CH - 1) // CH

BLK = 2048
EBLK = 32768

_f32 = jnp.float32
_i32 = jnp.int32


def _k1a_body(x_ref, w_ref, as_ref, ad_ref, h_ref, s_ref, t_ref):
    h = jnp.dot(x_ref[...], w_ref[...], preferred_element_type=_f32)
    h_ref[...] = h
    s_ref[...] = jnp.sum(h * as_ref[...][None, :], axis=1)
    t_ref[...] = jnp.sum(h * ad_ref[...][None, :], axis=1)


_k1a = pl.pallas_call(
    _k1a_body,
    grid=(NP // BLK,),
    in_specs=[
        pl.BlockSpec((BLK, D), lambda i: (i, 0)),
        pl.BlockSpec((D, D), lambda i: (0, 0)),
        pl.BlockSpec((D,), lambda i: (0,)),
        pl.BlockSpec((D,), lambda i: (0,)),
    ],
    out_specs=[
        pl.BlockSpec((BLK, D), lambda i: (i, 0)),
        pl.BlockSpec((BLK,), lambda i: (i,)),
        pl.BlockSpec((BLK,), lambda i: (i,)),
    ],
    out_shape=[
        jax.ShapeDtypeStruct((NP, D), _f32),
        jax.ShapeDtypeStruct((NP,), _f32),
        jax.ShapeDtypeStruct((NP,), _f32),
    ],
)


E8 = E // 8
EBLK8 = 5000


def _k1b_body(ea_ref, w1_ref, w2_ref, u1_ref, u2_ref):
    ea = ea_ref[...]
    u1_ref[...] = jnp.dot(ea, w1_ref[...], preferred_element_type=_f32)
    u2_ref[...] = jnp.dot(ea, w2_ref[...], preferred_element_type=_f32)


_k1b = pl.pallas_call(
    _k1b_body,
    grid=(E8 // EBLK8,),
    in_specs=[
        pl.BlockSpec((EBLK8, D), lambda i: (i, 0)),
        pl.BlockSpec((D, 8), lambda i: (0, 0)),
        pl.BlockSpec((D, 8), lambda i: (0, 0)),
    ],
    out_specs=[
        pl.BlockSpec((EBLK8, 8), lambda i: (i, 0)),
        pl.BlockSpec((EBLK8, 8), lambda i: (i, 0)),
    ],
    out_shape=[
        jax.ShapeDtypeStruct((E8, 8), _f32),
        jax.ShapeDtypeStruct((E8, 8), _f32),
    ],
)


_mesh = plsc.VectorSubcoreMesh(
    core_axis_name="c", subcore_axis_name="s", num_cores=NC, num_subcores=NS
)

_sc_params = pltpu.CompilerParams(needs_layout_passes=False)


@functools.partial(
    pl.kernel,
    out_type=[
        jax.ShapeDtypeStruct((NP, D), _f32),
        jax.ShapeDtypeStruct((NP, D), _f32),
        jax.ShapeDtypeStruct((NW, NP), _f32),
    ],
    mesh=_mesh,
    scratch_types=[
        pltpu.VMEM((SB,), _i32),
        pltpu.VMEM((SB,), _i32),
        pltpu.VMEM((SB,), _f32),
        pltpu.VMEM((NP,), _f32),
        pltpu.VMEM((NP,), _f32),
        pltpu.VMEM((CH,), _i32),
        pltpu.VMEM((CH,), _i32),
        pltpu.VMEM((CH,), _f32),
        pltpu.VMEM((CH, D), _f32),
        pltpu.VMEM((1, NP), _f32),
        pltpu.VMEM_SHARED((N, D), _f32),
        pltpu.SemaphoreType.DMA,
    ],
    compiler_params=_sc_params,
)
def _l1(src_hbm, dst_hbm, u_hbm, s_hbm, t_hbm, h_hbm,
        numa, numb, dend,
        src_v, dst_v, u_v, s_v, t_v, idx_s, idx_d, exq, rows, den_v,
        accn, sem):
    cc = lax.axis_index("c")
    sid = lax.axis_index("s")
    wid = sid * NC + cc
    iota = lax.iota(_i32, 16)
    zeros16 = jnp.zeros((16,), _f32)

    def _zrows(k, _):
        e = k * 16 + iota
        plsc.store_scatter(rows, [e // D, e % D], zeros16)
        return _
    lax.fori_loop(0, CH * D // 16, _zrows, None)

    def _zden(k, _):
        den_v[0, pl.ds(k * 16, 16)] = zeros16
        return _
    lax.fori_loop(0, NP // 16, _zden, None)

    for j in range(pl.cdiv(NDR, NS)):
        grp = sid + NS * j

        def _zz():
            pltpu.sync_copy(rows, accn.at[pl.ds(grp * CH, CH)])

        if (j + 1) * NS <= NDR:
            _zz()
        else:
            pl.when(grp < NDR)(_zz)
    plsc.subcore_barrier()

    pltpu.sync_copy(s_hbm, s_v)
    pltpu.sync_copy(t_hbm, t_v)

    base = wid * EPW

    def _block(b, _):
        boff = base + b * SB
        pltpu.sync_copy(src_hbm.at[pl.ds(boff, SB)], src_v)
        pltpu.sync_copy(dst_hbm.at[pl.ds(boff, SB)], dst_v)
        pltpu.sync_copy(u_hbm.at[pl.ds(boff, SB)], u_v)

        def _chunk(c, _):
            off = c * CH
            cp = pltpu.async_copy(h_hbm.at[src_v.at[pl.ds(off, CH)]], rows, sem)
            for k in range(CH // 16):
                id16 = dst_v[pl.ds(off + k * 16, 16)]
                sv = plsc.load_gather(s_v, [src_v[pl.ds(off + k * 16, 16)]])
                tv = plsc.load_gather(t_v, [id16])
                uv = u_v[pl.ds(off + k * 16, 16)]
                a = sv + tv + uv
                a = jnp.where(a > 0, a, 0.2 * a)
                ex = jnp.exp(a)
                exq[pl.ds(k * 16, 16)] = ex
                plsc.addupdate_scatter(den_v, [jnp.zeros((16,), _i32), id16], ex)
            cp.wait()
            for k in range(CH // 16):
                rid = k * 16 + iota
                ex = exq[pl.ds(k * 16, 16)]

                def _feat(f, _):
                    fv = jnp.full((16,), f, _i32)
                    col = plsc.load_gather(rows, [rid, fv])
                    plsc.store_scatter(rows, [rid, fv], col * ex)
                    return _
                lax.fori_loop(0, D, _feat, None, unroll=8)
            pltpu.sync_copy(rows, accn.at[dst_v.at[pl.ds(off, CH)]], add=True)
            return _
        lax.fori_loop(0, NCC, _chunk, None)
        return _
    lax.fori_loop(0, NCB, _block, None)

    pltpu.sync_copy(den_v, dend.at[pl.ds(wid, 1)])

    plsc.subcore_barrier()

    for j in range(pl.cdiv(NDR, NS)):
        grp = sid + NS * j

        def _drain():
            @pl.when(cc == 0)
            def _():
                pltpu.sync_copy(accn.at[pl.ds(grp * CH, CH)],
                                numa.at[pl.ds(grp * CH, CH)])

            @pl.when(cc == 1)
            def _():
                pltpu.sync_copy(accn.at[pl.ds(grp * CH, CH)],
                                numb.at[pl.ds(grp * CH, CH)])

        if (j + 1) * NS <= NDR:
            _drain()
        else:
            pl.when(grp < NDR)(_drain)


def _k4_body(na_ref, nb_ref, dd_ref, b1_ref, w2_ref, g_ref):
    num = na_ref[...] + nb_ref[...]
    den = jnp.sum(dd_ref[...], axis=0)
    h2 = jnp.maximum(num / (den[:, None] + 1e-16) + b1_ref[...][None, :], 0.0)
    g_ref[...] = jnp.sum(h2 * w2_ref[...][None, :], axis=1)


_k4 = pl.pallas_call(
    _k4_body,
    grid=(NP // BLK,),
    in_specs=[
        pl.BlockSpec((BLK, D), lambda i: (i, 0)),
        pl.BlockSpec((BLK, D), lambda i: (i, 0)),
        pl.BlockSpec((NW, BLK), lambda i: (0, i)),
        pl.BlockSpec((D,), lambda i: (0,)),
        pl.BlockSpec((D,), lambda i: (0,)),
    ],
    out_specs=[pl.BlockSpec((BLK,), lambda i: (i,))],
    out_shape=[jax.ShapeDtypeStruct((NP,), _f32)],
)


@functools.partial(
    pl.kernel,
    out_type=[
        jax.ShapeDtypeStruct((NW, NP), _f32),
        jax.ShapeDtypeStruct((NW, NP), _f32),
    ],
    mesh=_mesh,
    scratch_types=[
        pltpu.VMEM((EPW,), _i32),
        pltpu.VMEM((EPW,), _i32),
        pltpu.VMEM((EPW,), _f32),
        pltpu.VMEM((NP,), _f32),
        pltpu.VMEM((16,), _f32),
        pltpu.VMEM((1, NP), _f32),
        pltpu.VMEM((1, NP), _f32),
    ],
    compiler_params=_sc_params,
)
def _l2(src_hbm, dst_hbm, u_hbm, g_hbm, prm_hbm,
        num2, den2,
        src_v, dst_v, u_v, g_v, prm_v, num_v, den_v):
    cc = lax.axis_index("c")
    sid = lax.axis_index("s")
    wid = sid * NC + cc
    iota = lax.iota(_i32, 16)
    zeros16 = jnp.zeros((16,), _f32)

    def _zacc(k, _):
        num_v[0, pl.ds(k * 16, 16)] = zeros16
        den_v[0, pl.ds(k * 16, 16)] = zeros16
        return _
    lax.fori_loop(0, NP // 16, _zacc, None)

    base = wid * EPW
    pltpu.sync_copy(src_hbm.at[pl.ds(base, EPW)], src_v)
    pltpu.sync_copy(dst_hbm.at[pl.ds(base, EPW)], dst_v)
    pltpu.sync_copy(u_hbm.at[pl.ds(base, EPW)], u_v)
    pltpu.sync_copy(g_hbm, g_v)
    pltpu.sync_copy(prm_hbm, prm_v)
    a2s = plsc.load_gather(prm_v, [jnp.zeros((16,), _i32)])
    a2d = plsc.load_gather(prm_v, [jnp.ones((16,), _i32)])

    def _chunk(c, _):
        off = c * 16
        is16 = src_v[pl.ds(off, 16)]
        id16 = dst_v[pl.ds(off, 16)]
        gs = plsc.load_gather(g_v, [is16])
        gd = plsc.load_gather(g_v, [id16])
        a = a2s * gs + a2d * gd + u_v[pl.ds(off, 16)]
        a = jnp.where(a > 0, a, 0.2 * a)
        ex = jnp.exp(a)
        plsc.addupdate_scatter(num_v, [jnp.zeros((16,), _i32), id16], ex * gs)
        plsc.addupdate_scatter(den_v, [jnp.zeros((16,), _i32), id16], ex)
        return _
    lax.fori_loop(0, EPW // 16, _chunk, None, unroll=4)

    pltpu.sync_copy(num_v, num2.at[pl.ds(wid, 1)])
    pltpu.sync_copy(den_v, den2.at[pl.ds(wid, 1)])


def _k6_body(n2_ref, d2_ref, b2_ref, o_ref):
    num = jnp.sum(n2_ref[...], axis=0)
    den = jnp.sum(d2_ref[...], axis=0)
    o_ref[...] = jax.nn.sigmoid(num / (den + 1e-16) + b2_ref[0, 0])


_k6 = pl.pallas_call(
    _k6_body,
    grid=(NP // BLK,),
    in_specs=[
        pl.BlockSpec((NW, BLK), lambda i: (0, i)),
        pl.BlockSpec((NW, BLK), lambda i: (0, i)),
        pl.BlockSpec((1, 1), lambda i: (0, 0)),
    ],
    out_specs=[pl.BlockSpec((BLK,), lambda i: (i,))],
    out_shape=[jax.ShapeDtypeStruct((NP,), _f32)],
)


def kernel(x, edge_index, edge_attr,
           W1, a_src1, a_dst1, We1, ae1, b1,
           W2, a_src2, a_dst2, We2, ae2, b2):
    src = edge_index[0].astype(_i32)
    dst = edge_index[1].astype(_i32)
    xp = jnp.pad(x, ((0, NP - N), (0, 0)))
    we1 = (We1 @ ae1).astype(_f32)
    we2 = (We2[:, 0] * ae2[0]).astype(_f32)
    w81 = jnp.kron(jnp.eye(8, dtype=_f32), we1[:, None])
    w82 = jnp.kron(jnp.eye(8, dtype=_f32), we2[:, None])
    ea8 = edge_attr.reshape(E8, D)
    w2v = W2[:, 0].astype(_f32)
    prm2 = jnp.concatenate([a_src2, a_dst2, jnp.zeros((14,), _f32)])

    h, s, t = _k1a(xp, W1, a_src1, a_dst1)
    u18, u28 = _k1b(ea8, w81, w82)
    u1 = u18.reshape(E)
    u2 = u28.reshape(E)
    numa, numb, dend = _l1(src, dst, u1, s, t, h)
    g, = _k4(numa, numb, dend, b1, w2v)
    n2, d2 = _l2(src, dst, u2, g, prm2)
    out, = _k6(n2, d2, b2.reshape(1, 1))
    return out[:N]


# speedup vs baseline: 38.8228x; 3.4820x over previous
import functools

import jax
import jax.numpy as jnp
from jax import lax
from jax.experimental import pallas as pl
from jax.experimental.pallas import tpu as pltpu
from jax.experimental.pallas import tpu_sc as plsc

N = 10000
NP = 10240
E = 320000
D = 128
NC = 2
NS = 16
NW = NC * NS
EPW = E // NW
CH = 80
SB = 400
NCB = EPW // SB
NCC = SB // CH
NDR = (N + CH - 1) // CH

BLK = 2048
EBLK = 32768

_f32 = jnp.float32
_i32 = jnp.int32


def _k1a_body(x_ref, w_ref, as_ref, ad_ref, h_ref, s_ref, t_ref):
    h = jnp.dot(x_ref[...], w_ref[...], preferred_element_type=_f32)
    h_ref[...] = h
    s_ref[...] = jnp.sum(h * as_ref[...][None, :], axis=1)
    t_ref[...] = jnp.sum(h * ad_ref[...][None, :], axis=1)


_k1a = pl.pallas_call(
    _k1a_body,
    grid=(NP // BLK,),
    in_specs=[
        pl.BlockSpec((BLK, D), lambda i: (i, 0)),
        pl.BlockSpec((D, D), lambda i: (0, 0)),
        pl.BlockSpec((D,), lambda i: (0,)),
        pl.BlockSpec((D,), lambda i: (0,)),
    ],
    out_specs=[
        pl.BlockSpec((BLK, D), lambda i: (i, 0)),
        pl.BlockSpec((BLK,), lambda i: (i,)),
        pl.BlockSpec((BLK,), lambda i: (i,)),
    ],
    out_shape=[
        jax.ShapeDtypeStruct((NP, D), _f32),
        jax.ShapeDtypeStruct((NP,), _f32),
        jax.ShapeDtypeStruct((NP,), _f32),
    ],
)


E8 = E // 8
EBLK8 = 5000


def _k1b_body(ea_ref, w1_ref, w2_ref, u1_ref, u2_ref):
    ea = ea_ref[...]
    u1_ref[...] = jnp.dot(ea, w1_ref[...], preferred_element_type=_f32)
    u2_ref[...] = jnp.dot(ea, w2_ref[...], preferred_element_type=_f32)


_k1b = pl.pallas_call(
    _k1b_body,
    grid=(E8 // EBLK8,),
    in_specs=[
        pl.BlockSpec((EBLK8, D), lambda i: (i, 0)),
        pl.BlockSpec((D, 8), lambda i: (0, 0)),
        pl.BlockSpec((D, 8), lambda i: (0, 0)),
    ],
    out_specs=[
        pl.BlockSpec((EBLK8, 8), lambda i: (i, 0)),
        pl.BlockSpec((EBLK8, 8), lambda i: (i, 0)),
    ],
    out_shape=[
        jax.ShapeDtypeStruct((E8, 8), _f32),
        jax.ShapeDtypeStruct((E8, 8), _f32),
    ],
)


_mesh = plsc.VectorSubcoreMesh(
    core_axis_name="c", subcore_axis_name="s", num_cores=NC, num_subcores=NS
)

_sc_params = pltpu.CompilerParams(needs_layout_passes=False)


@functools.partial(
    pl.kernel,
    out_type=[
        jax.ShapeDtypeStruct((NP, D), _f32),
        jax.ShapeDtypeStruct((NP, D), _f32),
        jax.ShapeDtypeStruct((NW, NP), _f32),
    ],
    mesh=_mesh,
    scratch_types=[
        pltpu.VMEM((SB,), _i32),
        pltpu.VMEM((SB,), _i32),
        pltpu.VMEM((SB,), _f32),
        pltpu.VMEM((NP,), _f32),
        pltpu.VMEM((NP,), _f32),
        pltpu.VMEM((CH,), _i32),
        pltpu.VMEM((CH,), _i32),
        pltpu.VMEM((CH,), _f32),
        pltpu.VMEM((CH, D), _f32),
        pltpu.VMEM((1, NP), _f32),
        pltpu.VMEM_SHARED((N, D), _f32),
        pltpu.SemaphoreType.DMA,
    ],
    compiler_params=_sc_params,
)
def _l1(src_hbm, dst_hbm, u_hbm, s_hbm, t_hbm, h_hbm,
        numa, numb, dend,
        src_v, dst_v, u_v, s_v, t_v, idx_s, idx_d, exq, rows, den_v,
        accn, sem):
    cc = lax.axis_index("c")
    sid = lax.axis_index("s")
    wid = sid * NC + cc
    iota = lax.iota(_i32, 16)
    zeros16 = jnp.zeros((16,), _f32)

    def _zrows(k, _):
        e = k * 16 + iota
        plsc.store_scatter(rows, [e // D, e % D], zeros16)
        return _
    lax.fori_loop(0, CH * D // 16, _zrows, None)

    def _zden(k, _):
        den_v[0, pl.ds(k * 16, 16)] = zeros16
        return _
    lax.fori_loop(0, NP // 16, _zden, None)

    for j in range(pl.cdiv(NDR, NS)):
        grp = sid + NS * j

        def _zz():
            pltpu.sync_copy(rows, accn.at[pl.ds(grp * CH, CH)])

        if (j + 1) * NS <= NDR:
            _zz()
        else:
            pl.when(grp < NDR)(_zz)
    plsc.subcore_barrier()

    pltpu.sync_copy(s_hbm, s_v)
    pltpu.sync_copy(t_hbm, t_v)

    base = wid * EPW

    def _block(b, _):
        boff = base + b * SB
        pltpu.sync_copy(src_hbm.at[pl.ds(boff, SB)], src_v)
        pltpu.sync_copy(dst_hbm.at[pl.ds(boff, SB)], dst_v)
        pltpu.sync_copy(u_hbm.at[pl.ds(boff, SB)], u_v)

        def _chunk(c, _):
            off = c * CH
            cp = pltpu.async_copy(h_hbm.at[src_v.at[pl.ds(off, CH)]], rows, sem)
            for k in range(CH // 16):
                id16 = dst_v[pl.ds(off + k * 16, 16)]
                sv = plsc.load_gather(s_v, [src_v[pl.ds(off + k * 16, 16)]])
                tv = plsc.load_gather(t_v, [id16])
                uv = u_v[pl.ds(off + k * 16, 16)]
                a = sv + tv + uv
                a = jnp.where(a > 0, a, 0.2 * a)
                ex = jnp.exp(a)
                exq[pl.ds(k * 16, 16)] = ex
                plsc.addupdate_scatter(den_v, [jnp.zeros((16,), _i32), id16], ex)
            cp.wait()
            for e in range(CH):
                splat = plsc.load_gather(exq, [jnp.full((16,), e, _i32)])
                for f8 in range(D // 16):
                    sl = pl.ds(f8 * 16, 16)
                    rows[e, sl] = rows[e, sl] * splat
            pltpu.sync_copy(rows, accn.at[dst_v.at[pl.ds(off, CH)]], add=True)
            return _
        lax.fori_loop(0, NCC, _chunk, None)
        return _
    lax.fori_loop(0, NCB, _block, None)

    pltpu.sync_copy(den_v, dend.at[pl.ds(wid, 1)])

    plsc.subcore_barrier()

    for j in range(pl.cdiv(NDR, NS)):
        grp = sid + NS * j

        def _drain():
            @pl.when(cc == 0)
            def _():
                pltpu.sync_copy(accn.at[pl.ds(grp * CH, CH)],
                                numa.at[pl.ds(grp * CH, CH)])

            @pl.when(cc == 1)
            def _():
                pltpu.sync_copy(accn.at[pl.ds(grp * CH, CH)],
                                numb.at[pl.ds(grp * CH, CH)])

        if (j + 1) * NS <= NDR:
            _drain()
        else:
            pl.when(grp < NDR)(_drain)


def _k4_body(na_ref, nb_ref, dd_ref, b1_ref, w2_ref, g_ref):
    num = na_ref[...] + nb_ref[...]
    den = jnp.sum(dd_ref[...], axis=0)
    h2 = jnp.maximum(num / (den[:, None] + 1e-16) + b1_ref[...][None, :], 0.0)
    g_ref[...] = jnp.sum(h2 * w2_ref[...][None, :], axis=1)


_k4 = pl.pallas_call(
    _k4_body,
    grid=(NP // BLK,),
    in_specs=[
        pl.BlockSpec((BLK, D), lambda i: (i, 0)),
        pl.BlockSpec((BLK, D), lambda i: (i, 0)),
        pl.BlockSpec((NW, BLK), lambda i: (0, i)),
        pl.BlockSpec((D,), lambda i: (0,)),
        pl.BlockSpec((D,), lambda i: (0,)),
    ],
    out_specs=[pl.BlockSpec((BLK,), lambda i: (i,))],
    out_shape=[jax.ShapeDtypeStruct((NP,), _f32)],
)


@functools.partial(
    pl.kernel,
    out_type=[
        jax.ShapeDtypeStruct((NW, NP), _f32),
        jax.ShapeDtypeStruct((NW, NP), _f32),
    ],
    mesh=_mesh,
    scratch_types=[
        pltpu.VMEM((EPW,), _i32),
        pltpu.VMEM((EPW,), _i32),
        pltpu.VMEM((EPW,), _f32),
        pltpu.VMEM((NP,), _f32),
        pltpu.VMEM((16,), _f32),
        pltpu.VMEM((1, NP), _f32),
        pltpu.VMEM((1, NP), _f32),
    ],
    compiler_params=_sc_params,
)
def _l2(src_hbm, dst_hbm, u_hbm, g_hbm, prm_hbm,
        num2, den2,
        src_v, dst_v, u_v, g_v, prm_v, num_v, den_v):
    cc = lax.axis_index("c")
    sid = lax.axis_index("s")
    wid = sid * NC + cc
    iota = lax.iota(_i32, 16)
    zeros16 = jnp.zeros((16,), _f32)

    def _zacc(k, _):
        num_v[0, pl.ds(k * 16, 16)] = zeros16
        den_v[0, pl.ds(k * 16, 16)] = zeros16
        return _
    lax.fori_loop(0, NP // 16, _zacc, None)

    base = wid * EPW
    pltpu.sync_copy(src_hbm.at[pl.ds(base, EPW)], src_v)
    pltpu.sync_copy(dst_hbm.at[pl.ds(base, EPW)], dst_v)
    pltpu.sync_copy(u_hbm.at[pl.ds(base, EPW)], u_v)
    pltpu.sync_copy(g_hbm, g_v)
    pltpu.sync_copy(prm_hbm, prm_v)
    a2s = plsc.load_gather(prm_v, [jnp.zeros((16,), _i32)])
    a2d = plsc.load_gather(prm_v, [jnp.ones((16,), _i32)])

    def _chunk(c, _):
        off = c * 16
        is16 = src_v[pl.ds(off, 16)]
        id16 = dst_v[pl.ds(off, 16)]
        gs = plsc.load_gather(g_v, [is16])
        gd = plsc.load_gather(g_v, [id16])
        a = a2s * gs + a2d * gd + u_v[pl.ds(off, 16)]
        a = jnp.where(a > 0, a, 0.2 * a)
        ex = jnp.exp(a)
        plsc.addupdate_scatter(num_v, [jnp.zeros((16,), _i32), id16], ex * gs)
        plsc.addupdate_scatter(den_v, [jnp.zeros((16,), _i32), id16], ex)
        return _
    lax.fori_loop(0, EPW // 16, _chunk, None, unroll=4)

    pltpu.sync_copy(num_v, num2.at[pl.ds(wid, 1)])
    pltpu.sync_copy(den_v, den2.at[pl.ds(wid, 1)])


def _k6_body(n2_ref, d2_ref, b2_ref, o_ref):
    num = jnp.sum(n2_ref[...], axis=0)
    den = jnp.sum(d2_ref[...], axis=0)
    o_ref[...] = jax.nn.sigmoid(num / (den + 1e-16) + b2_ref[0, 0])


_k6 = pl.pallas_call(
    _k6_body,
    grid=(NP // BLK,),
    in_specs=[
        pl.BlockSpec((NW, BLK), lambda i: (0, i)),
        pl.BlockSpec((NW, BLK), lambda i: (0, i)),
        pl.BlockSpec((1, 1), lambda i: (0, 0)),
    ],
    out_specs=[pl.BlockSpec((BLK,), lambda i: (i,))],
    out_shape=[jax.ShapeDtypeStruct((NP,), _f32)],
)


def kernel(x, edge_index, edge_attr,
           W1, a_src1, a_dst1, We1, ae1, b1,
           W2, a_src2, a_dst2, We2, ae2, b2):
    src = edge_index[0].astype(_i32)
    dst = edge_index[1].astype(_i32)
    xp = jnp.pad(x, ((0, NP - N), (0, 0)))
    we1 = (We1 @ ae1).astype(_f32)
    we2 = (We2[:, 0] * ae2[0]).astype(_f32)
    w81 = jnp.kron(jnp.eye(8, dtype=_f32), we1[:, None])
    w82 = jnp.kron(jnp.eye(8, dtype=_f32), we2[:, None])
    ea8 = edge_attr.reshape(E8, D)
    w2v = W2[:, 0].astype(_f32)
    prm2 = jnp.concatenate([a_src2, a_dst2, jnp.zeros((14,), _f32)])

    h, s, t = _k1a(xp, W1, a_src1, a_dst1)
    u18, u28 = _k1b(ea8, w81, w82)
    u1 = u18.reshape(E)
    u2 = u28.reshape(E)
    numa, numb, dend = _l1(src, dst, u1, s, t, h)
    g, = _k4(numa, numb, dend, b1, w2v)
    n2, d2 = _l2(src, dst, u2, g, prm2)
    out, = _k6(n2, d2, b2.reshape(1, 1))
    return out[:N]
